# initial kernel scaffold (unmeasured)
import jax
import jax.numpy as jnp
from jax import lax
from jax.experimental import pallas as pl
from jax.experimental.pallas import tpu as pltpu


def kernel(
    x,
):
    def body(*refs):
        pass

    out_shape = jax.ShapeDtypeStruct(..., jnp.float32)
    return pl.pallas_call(body, out_shape=out_shape)(...)


# baseline (device time: 21125 ns/iter reference)
import jax
import jax.numpy as jnp
from jax import lax
from jax.experimental import pallas as pl
from jax.experimental.pallas import tpu as pltpu

N_DEV = 8


def kernel(x):
    m, n = x.shape

    def body(x_ref, out_ref, totals_ref, send_sems, recv_sems):
        my_pos = lax.axis_index("i")
        right = lax.rem(my_pos + 1, N_DEV)

        y = x_ref[:, :].astype(jnp.float32)
        s = 1
        while s < m:
            shifted = jnp.concatenate(
                [jnp.ones((s, n), jnp.float32), y[: m - s, :]], axis=0
            )
            y = y * shifted
            s *= 2

        totals_ref[0:1, :] = y[m - 1 : m, :]
        for h in range(N_DEV - 1):
            rdma = pltpu.make_async_remote_copy(
                src_ref=totals_ref.at[pl.ds(h, 1)],
                dst_ref=totals_ref.at[pl.ds(h + 1, 1)],
                send_sem=send_sems.at[h],
                recv_sem=recv_sems.at[h],
                device_id=(right,),
                device_id_type=pl.DeviceIdType.MESH,
            )
            rdma.start()
            rdma.wait()

        carry = jnp.ones((1, n), jnp.float32)
        for k in range(1, N_DEV):
            row = totals_ref[k : k + 1, :]
            carry = carry * jnp.where(k <= my_pos, row, 1.0)

        out_ref[:, :] = y * carry

    return pl.pallas_call(
        body,
        out_shape=jax.ShapeDtypeStruct((m, n), jnp.float32),
        in_specs=[pl.BlockSpec(memory_space=pltpu.VMEM)],
        out_specs=pl.BlockSpec(memory_space=pltpu.VMEM),
        scratch_shapes=[
            pltpu.VMEM((N_DEV, n), jnp.float32),
            pltpu.SemaphoreType.DMA((N_DEV - 1,)),
            pltpu.SemaphoreType.DMA((N_DEV - 1,)),
        ],
    )(x)


# device time: 8623 ns/iter; 2.4498x vs baseline; 2.4498x over previous
import jax
import jax.numpy as jnp
from jax import lax
from jax.experimental import pallas as pl
from jax.experimental.pallas import tpu as pltpu

N_DEV = 8


def kernel(x):
    m, n = x.shape

    def body(x_ref, out_ref, total_ref, recv_buf, send_sems, recv_sems):
        my_pos = lax.axis_index("i")

        xf = x_ref[:, :].astype(jnp.float32)

        t = xf
        rows = m
        while rows > 1:
            half = rows // 2
            t = t[:half, :] * t[half:rows, :]
            rows = half
        total_ref[0:1, :] = t[0:1, :]

        for d in range(1, N_DEV):
            @pl.when(my_pos + d < N_DEV)
            def _(d=d):
                rdma = pltpu.make_async_remote_copy(
                    src_ref=total_ref.at[pl.ds(0, 1)],
                    dst_ref=recv_buf.at[pl.ds(d, 1)],
                    send_sem=send_sems.at[d - 1],
                    recv_sem=recv_sems.at[d - 1],
                    device_id=(my_pos + d,),
                    device_id_type=pl.DeviceIdType.MESH,
                )
                rdma.start()

        y = xf
        s = 1
        while s < m:
            shifted = jnp.concatenate(
                [jnp.ones((s, n), jnp.float32), y[: m - s, :]], axis=0
            )
            y = y * shifted
            s *= 2

        carry = jnp.ones((1, n), jnp.float32)
        for d in range(1, N_DEV):
            @pl.when(my_pos - d >= 0)
            def _(d=d):
                recv = pltpu.make_async_remote_copy(
                    src_ref=total_ref.at[pl.ds(0, 1)],
                    dst_ref=recv_buf.at[pl.ds(d, 1)],
                    send_sem=send_sems.at[d - 1],
                    recv_sem=recv_sems.at[d - 1],
                    device_id=(my_pos - d,),
                    device_id_type=pl.DeviceIdType.MESH,
                )
                recv.wait_recv()

            row = recv_buf[d : d + 1, :]
            carry = carry * jnp.where(d <= my_pos, row, 1.0)

        out_ref[:, :] = y * carry

        for d in range(1, N_DEV):
            @pl.when(my_pos + d < N_DEV)
            def _(d=d):
                send = pltpu.make_async_remote_copy(
                    src_ref=total_ref.at[pl.ds(0, 1)],
                    dst_ref=recv_buf.at[pl.ds(d, 1)],
                    send_sem=send_sems.at[d - 1],
                    recv_sem=recv_sems.at[d - 1],
                    device_id=(my_pos + d,),
                    device_id_type=pl.DeviceIdType.MESH,
                )
                send.wait_send()

    return pl.pallas_call(
        body,
        out_shape=jax.ShapeDtypeStruct((m, n), jnp.float32),
        in_specs=[pl.BlockSpec(memory_space=pltpu.VMEM)],
        out_specs=pl.BlockSpec(memory_space=pltpu.VMEM),
        scratch_shapes=[
            pltpu.VMEM((1, n), jnp.float32),
            pltpu.VMEM((N_DEV, n), jnp.float32),
            pltpu.SemaphoreType.DMA((N_DEV - 1,)),
            pltpu.SemaphoreType.DMA((N_DEV - 1,)),
        ],
    )(x)


# device time: 7439 ns/iter; 2.8398x vs baseline; 1.1592x over previous
import jax
import jax.numpy as jnp
from jax import lax
from jax.experimental import pallas as pl
from jax.experimental.pallas import tpu as pltpu

N_DEV = 8


def kernel(x):
    m, n = x.shape

    def body(x_ref, out_ref, total_ref, recv_buf, send_sems, recv_sems):
        my_pos = lax.axis_index("i")

        bar = pltpu.get_barrier_semaphore()
        for d in range(1, N_DEV):
            @pl.when(my_pos - d >= 0)
            def _(d=d):
                pl.semaphore_signal(
                    bar, inc=1,
                    device_id=(my_pos - d,),
                    device_id_type=pl.DeviceIdType.MESH,
                )

        xf = x_ref[:, :].astype(jnp.float32)
        t = xf
        rows = m
        while rows > 1:
            half = rows // 2
            t = t[:half, :] * t[half:rows, :]
            rows = half
        total_ref[0:1, :] = t[0:1, :]

        pl.semaphore_wait(bar, N_DEV - 1 - my_pos)
        for d in range(1, N_DEV):
            @pl.when(my_pos + d < N_DEV)
            def _(d=d):
                rdma = pltpu.make_async_remote_copy(
                    src_ref=total_ref.at[pl.ds(0, 1)],
                    dst_ref=recv_buf.at[pl.ds(d, 1)],
                    send_sem=send_sems.at[d - 1],
                    recv_sem=recv_sems.at[d - 1],
                    device_id=(my_pos + d,),
                    device_id_type=pl.DeviceIdType.MESH,
                )
                rdma.start()

        y = xf
        s = 1
        while s < m:
            shifted = jnp.concatenate(
                [jnp.ones((s, n), jnp.float32), y[: m - s, :]], axis=0
            )
            y = y * shifted
            s *= 2

        carry = jnp.ones((1, n), jnp.float32)
        for d in range(1, N_DEV):
            @pl.when(my_pos - d >= 0)
            def _(d=d):
                recv = pltpu.make_async_remote_copy(
                    src_ref=total_ref.at[pl.ds(0, 1)],
                    dst_ref=recv_buf.at[pl.ds(d, 1)],
                    send_sem=send_sems.at[d - 1],
                    recv_sem=recv_sems.at[d - 1],
                    device_id=(my_pos - d,),
                    device_id_type=pl.DeviceIdType.MESH,
                )
                recv.wait_recv()

            row = recv_buf[d : d + 1, :]
            carry = carry * jnp.where(d <= my_pos, row, 1.0)

        out_ref[:, :] = y * carry

        for d in range(1, N_DEV):
            @pl.when(my_pos + d < N_DEV)
            def _(d=d):
                send = pltpu.make_async_remote_copy(
                    src_ref=total_ref.at[pl.ds(0, 1)],
                    dst_ref=recv_buf.at[pl.ds(d, 1)],
                    send_sem=send_sems.at[d - 1],
                    recv_sem=recv_sems.at[d - 1],
                    device_id=(my_pos + d,),
                    device_id_type=pl.DeviceIdType.MESH,
                )
                send.wait_send()

    return pl.pallas_call(
        body,
        out_shape=jax.ShapeDtypeStruct((m, n), jnp.float32),
        in_specs=[pl.BlockSpec(memory_space=pltpu.VMEM)],
        out_specs=pl.BlockSpec(memory_space=pltpu.VMEM),
        scratch_shapes=[
            pltpu.VMEM((1, n), jnp.float32),
            pltpu.VMEM((N_DEV, n), jnp.float32),
            pltpu.SemaphoreType.DMA((N_DEV - 1,)),
            pltpu.SemaphoreType.DMA((N_DEV - 1,)),
        ],
        compiler_params=pltpu.CompilerParams(collective_id=0),
    )(x)


# device time: 7431 ns/iter; 2.8428x vs baseline; 1.0011x over previous
import jax
import jax.numpy as jnp
from jax import lax
from jax.experimental import pallas as pl
from jax.experimental.pallas import tpu as pltpu

N_DEV = 8


def kernel(x):
    m, n = x.shape

    def body(x_ref, out_ref, total_ref, recv_buf, send_sems, recv_sems):
        my_pos = lax.axis_index("i")

        bar = pltpu.get_barrier_semaphore()
        for d in range(1, N_DEV):
            @pl.when(my_pos - d >= 0)
            def _(d=d):
                pl.semaphore_signal(
                    bar, inc=1,
                    device_id=(my_pos - d,),
                    device_id_type=pl.DeviceIdType.MESH,
                )

        xf = x_ref[:, :].astype(jnp.float32)
        t = xf
        rows = m
        while rows > 1:
            half = rows // 2
            t = t[:half, :] * t[half:rows, :]
            rows = half
        total_ref[0:1, :] = t[0:1, :]

        for d in range(1, N_DEV):
            @pl.when(my_pos + d < N_DEV)
            def _(d=d):
                pl.semaphore_wait(bar, 1)
        for d in range(1, N_DEV):
            @pl.when(my_pos + d < N_DEV)
            def _(d=d):
                rdma = pltpu.make_async_remote_copy(
                    src_ref=total_ref.at[pl.ds(0, 1)],
                    dst_ref=recv_buf.at[pl.ds(d, 1)],
                    send_sem=send_sems.at[d - 1],
                    recv_sem=recv_sems.at[d - 1],
                    device_id=(my_pos + d,),
                    device_id_type=pl.DeviceIdType.MESH,
                )
                rdma.start()

        y = xf
        s = 1
        while s < m:
            shifted = jnp.concatenate(
                [jnp.ones((s, n), jnp.float32), y[: m - s, :]], axis=0
            )
            y = y * shifted
            s *= 2

        carry = jnp.ones((1, n), jnp.float32)
        for d in range(1, N_DEV):
            @pl.when(my_pos - d >= 0)
            def _(d=d):
                recv = pltpu.make_async_remote_copy(
                    src_ref=total_ref.at[pl.ds(0, 1)],
                    dst_ref=recv_buf.at[pl.ds(d, 1)],
                    send_sem=send_sems.at[d - 1],
                    recv_sem=recv_sems.at[d - 1],
                    device_id=(my_pos - d,),
                    device_id_type=pl.DeviceIdType.MESH,
                )
                recv.wait_recv()

            row = recv_buf[d : d + 1, :]
            carry = carry * jnp.where(d <= my_pos, row, 1.0)

        out_ref[:, :] = y * carry

        for d in range(1, N_DEV):
            @pl.when(my_pos + d < N_DEV)
            def _(d=d):
                send = pltpu.make_async_remote_copy(
                    src_ref=total_ref.at[pl.ds(0, 1)],
                    dst_ref=recv_buf.at[pl.ds(d, 1)],
                    send_sem=send_sems.at[d - 1],
                    recv_sem=recv_sems.at[d - 1],
                    device_id=(my_pos + d,),
                    device_id_type=pl.DeviceIdType.MESH,
                )
                send.wait_send()

    return pl.pallas_call(
        body,
        out_shape=jax.ShapeDtypeStruct((m, n), jnp.float32),
        in_specs=[pl.BlockSpec(memory_space=pltpu.VMEM)],
        out_specs=pl.BlockSpec(memory_space=pltpu.VMEM),
        scratch_shapes=[
            pltpu.VMEM((1, n), jnp.float32),
            pltpu.VMEM((N_DEV, n), jnp.float32),
            pltpu.SemaphoreType.DMA((N_DEV - 1,)),
            pltpu.SemaphoreType.DMA((N_DEV - 1,)),
        ],
        compiler_params=pltpu.CompilerParams(collective_id=0),
    )(x)
